# Initial kernel scaffold; baseline (speedup 1.0000x reference)
import jax
import jax.numpy as jnp
from jax.experimental import pallas as pl


def kernel(x, lens, embedding_weight):
    raise NotImplementedError("write your pallas kernel here")


# speedup vs baseline: 3.5393x; 3.5393x over previous
import functools

import jax
import jax.numpy as jnp
from jax import lax
from jax.experimental import pallas as pl
from jax.experimental.pallas import tpu as pltpu
from jax.experimental.pallas import tpu_sc as plsc

_G = 128


@functools.lru_cache(maxsize=None)
def _build_gather(V, D, N):
    info = plsc.get_sparse_core_info()
    NC, NS = info.num_cores, info.num_subcores
    NW = NC * NS
    assert N % (NW * _G) == 0
    per_w = N // NW
    n_chunks = per_w // _G
    mesh = plsc.VectorSubcoreMesh(core_axis_name="c", subcore_axis_name="s")

    @functools.partial(
        pl.kernel,
        mesh=mesh,
        compiler_params=pltpu.CompilerParams(use_tc_tiling_on_sc=False),
        out_type=jax.ShapeDtypeStruct((N, D), jnp.float32),
        scratch_types=[
            pltpu.VMEM((n_chunks, _G), jnp.int32),
            pltpu.VMEM((_G, D), jnp.float32),
            pltpu.SemaphoreType.DMA,
        ],
    )
    def gather_kernel(idx_hbm, table_hbm, out_hbm, idx_v, rows_v, sem):
        wid = lax.axis_index("s") * NC + lax.axis_index("c")
        pltpu.sync_copy(idx_hbm.at[wid], idx_v)
        base = wid * per_w

        def body(j, carry):
            pltpu.async_copy(table_hbm.at[idx_v.at[j]], rows_v, sem).wait()
            pltpu.sync_copy(rows_v, out_hbm.at[pl.ds(base + j * _G, _G)])
            return carry

        lax.fori_loop(0, n_chunks, body, 0)

    def run(idx, table):
        return gather_kernel(idx.reshape(NW, n_chunks, _G), table)

    return run


def kernel(x, lens, embedding_weight):
    B, L = x.shape
    V, D = embedding_weight.shape
    N = B * L
    run = _build_gather(V, D, N)
    out = run(x.reshape(N).astype(jnp.int32), embedding_weight)
    return out.reshape(B, L, D)


# speedup vs baseline: 4.2539x; 1.2019x over previous
import functools

import jax
import jax.numpy as jnp
from jax import lax
from jax.experimental import pallas as pl
from jax.experimental.pallas import tpu as pltpu
from jax.experimental.pallas import tpu_sc as plsc

_G = 128
_NBUF = 8
_LA = 4


@functools.lru_cache(maxsize=None)
def _build_gather(V, D, N):
    info = plsc.get_sparse_core_info()
    NC, NS = info.num_cores, info.num_subcores
    NW = NC * NS
    assert N % (NW * _G * _NBUF) == 0
    per_w = N // NW
    n_chunks = per_w // _G
    n_groups = n_chunks // _NBUF
    mesh = plsc.VectorSubcoreMesh(core_axis_name="c", subcore_axis_name="s")

    @functools.partial(
        pl.kernel,
        mesh=mesh,
        compiler_params=pltpu.CompilerParams(use_tc_tiling_on_sc=False),
        out_type=jax.ShapeDtypeStruct((N, D), jnp.float32),
        scratch_types=[
            pltpu.VMEM((n_chunks, _G), jnp.int32),
            pltpu.VMEM((_NBUF, _G, D), jnp.float32),
        ]
        + [pltpu.SemaphoreType.DMA] * (2 * _NBUF),
    )
    def gather_kernel(idx_hbm, table_hbm, out_hbm, idx_v, rows, *sems):
        gs = sems[:_NBUF]
        os_ = sems[_NBUF:]
        wid = lax.axis_index("s") * NC + lax.axis_index("c")
        pltpu.sync_copy(idx_hbm.at[wid], idx_v)
        base = wid * per_w

        def fire_gather(j, b):
            pltpu.async_copy(table_hbm.at[idx_v.at[j]], rows.at[b], gs[b])

        def wait_gather(j, b):
            pltpu.make_async_copy(table_hbm.at[idx_v.at[j]], rows.at[b],
                                  gs[b]).wait()

        def out_slice(j):
            return out_hbm.at[pl.ds(base + j * _G, _G)]

        def fire_out(j, b):
            pltpu.async_copy(rows.at[b], out_slice(j), os_[b])

        def wait_out(j, b):
            pltpu.make_async_copy(rows.at[b], out_slice(j), os_[b]).wait()

        for c in range(_LA):
            fire_gather(c, c % _NBUF)
        for j in range(_NBUF):
            b = j % _NBUF
            bn = (j + _LA) % _NBUF
            if j >= _LA:
                wait_out(j - _LA, bn)
            fire_gather(j + _LA, bn)
            wait_gather(j, b)
            fire_out(j, b)

        def group_body(g, carry):
            jg = g * _NBUF
            for b in range(_NBUF):
                j = jg + b
                bn = (b + _LA) % _NBUF
                wait_out(j - _LA, bn)
                fire_gather(j + _LA, bn)
                wait_gather(j, b)
                fire_out(j, b)
            return carry

        lax.fori_loop(1, n_groups - 1, group_body, 0)

        jg = (n_groups - 1) * _NBUF
        for b in range(_NBUF):
            j = jg + b
            bn = (b + _LA) % _NBUF
            if j + _LA < n_chunks:
                wait_out(j - _LA, bn)
                fire_gather(j + _LA, bn)
            wait_gather(j, b)
            fire_out(j, b)
        for b in range(_NBUF):
            wait_out(jg + b, b)

    def run(idx, table):
        return gather_kernel(idx.reshape(NW, n_chunks, _G), table)

    return run


def kernel(x, lens, embedding_weight):
    B, L = x.shape
    V, D = embedding_weight.shape
    N = B * L
    run = _build_gather(V, D, N)
    out = run(x.reshape(N).astype(jnp.int32), embedding_weight)
    return out.reshape(B, L, D)


# speedup vs baseline: 4.2567x; 1.0007x over previous
import functools

import jax
import jax.numpy as jnp
from jax import lax
from jax.experimental import pallas as pl
from jax.experimental.pallas import tpu as pltpu
from jax.experimental.pallas import tpu_sc as plsc

_NBUF = 8
_LA = 4
_GA = 104


@functools.lru_cache(maxsize=None)
def _build_gather(V, D, B, L):
    info = plsc.get_sparse_core_info()
    NC, NS = info.num_cores, info.num_subcores
    NW = NC * NS
    GB = L - _GA
    assert 0 < GB <= 128 and GB % 8 == 0 and B % NW == 0
    b_per_w = B // NW
    n_chunks = 2 * b_per_w
    assert n_chunks % _NBUF == 0
    n_groups = n_chunks // _NBUF
    mesh = plsc.VectorSubcoreMesh(core_axis_name="c", subcore_axis_name="s")

    @functools.partial(
        pl.kernel,
        mesh=mesh,
        compiler_params=pltpu.CompilerParams(use_tc_tiling_on_sc=False),
        out_type=jax.ShapeDtypeStruct((B, L, D), jnp.float32),
        scratch_types=[
            pltpu.VMEM((b_per_w, L), jnp.int32),
            pltpu.VMEM((_NBUF, _GA, D), jnp.float32),
        ]
        + [pltpu.SemaphoreType.DMA] * (2 * _NBUF),
    )
    def gather_kernel(idx_hbm, table_hbm, out_hbm, idx_v, rows, *sems):
        gs = sems[:_NBUF]
        os_ = sems[_NBUF:]
        wid = lax.axis_index("s") * NC + lax.axis_index("c")
        pltpu.sync_copy(idx_hbm.at[wid], idx_v)
        b_base = wid * b_per_w

        def geom(j, p):
            return j // 2, (0, _GA) if p == 0 else (_GA, GB)

        def fire_gather(j, p, b):
            r, (l0, g) = geom(j, p)
            pltpu.async_copy(table_hbm.at[idx_v.at[r, pl.ds(l0, g)]],
                             rows.at[b, pl.ds(0, g)], gs[b])

        def wait_gather(j, p, b):
            r, (l0, g) = geom(j, p)
            pltpu.make_async_copy(table_hbm.at[idx_v.at[r, pl.ds(l0, g)]],
                                  rows.at[b, pl.ds(0, g)], gs[b]).wait()

        def fire_out(j, p, b):
            r, (l0, g) = geom(j, p)
            pltpu.async_copy(rows.at[b, pl.ds(0, g)],
                             out_hbm.at[b_base + r, pl.ds(l0, g)], os_[b])

        def wait_out(j, p, b):
            r, (l0, g) = geom(j, p)
            pltpu.make_async_copy(rows.at[b, pl.ds(0, g)],
                                  out_hbm.at[b_base + r, pl.ds(l0, g)],
                                  os_[b]).wait()

        for c in range(_LA):
            fire_gather(c, c % 2, c % _NBUF)
        for j in range(_NBUF):
            b = j % _NBUF
            bn = (j + _LA) % _NBUF
            p = j % 2
            if j >= _LA:
                wait_out(j - _LA, p, bn)
            fire_gather(j + _LA, p, bn)
            wait_gather(j, p, b)
            fire_out(j, p, b)

        def group_body(g, carry):
            jg = g * _NBUF
            for b in range(_NBUF):
                j = jg + b
                p = b % 2
                bn = (b + _LA) % _NBUF
                wait_out(j - _LA, p, bn)
                fire_gather(j + _LA, p, bn)
                wait_gather(j, p, b)
                fire_out(j, p, b)
            return carry

        lax.fori_loop(1, n_groups - 1, group_body, 0)

        jg = (n_groups - 1) * _NBUF
        for b in range(_NBUF):
            j = jg + b
            p = b % 2
            bn = (b + _LA) % _NBUF
            if j + _LA < n_chunks:
                wait_out(j - _LA, p, bn)
                fire_gather(j + _LA, p, bn)
            wait_gather(j, p, b)
            fire_out(j, p, b)
        for b in range(_NBUF):
            wait_out(jg + b, b % 2, b)

    def run(idx, table):
        return gather_kernel(idx.reshape(NW, b_per_w, L), table)

    return run


def kernel(x, lens, embedding_weight):
    B, L = x.shape
    V, D = embedding_weight.shape
    run = _build_gather(V, D, B, L)
    return run(x.astype(jnp.int32), embedding_weight)
